# baseline (speedup 1.0000x reference)
import functools

import jax
import jax.numpy as jnp
from jax import lax
from jax.experimental import pallas as pl
from jax.experimental.pallas import tpu as pltpu
from jax.experimental.pallas import tpu_sc as plsc

_V = 100000
_D = 128
_L = 200
_BLK = 2000
_NB = _V // _BLK


def _mean_body(idx_hbm, emb_hbm, out_hbm, idx_v, rows_v, acc_v, sem):
    wid = lax.axis_index("s") * 2 + lax.axis_index("c")

    @pl.when(wid == 0)
    def _():
        pltpu.sync_copy(idx_hbm, idx_v)
        cp0 = pltpu.async_copy(emb_hbm.at[idx_v.at[0]],
                               rows_v.at[pl.ds(0, _L // 2)], sem)
        cp1 = pltpu.async_copy(emb_hbm.at[idx_v.at[1]],
                               rows_v.at[pl.ds(_L // 2, _L // 2)], sem)
        cp0.wait()
        cp1.wait()

        for k in range(_D // 16):
            acc_v[pl.ds(k * 16, 16)] = jnp.zeros((16,), jnp.float32)

        def body(i, carry):
            for k in range(_D // 16):
                plsc.addupdate(acc_v.at[pl.ds(k * 16, 16)],
                               rows_v[i, pl.ds(k * 16, 16)])
            return carry

        lax.fori_loop(0, _L, body, 0)

        scale = jnp.float32(1.0 / _L)
        for k in range(_D // 16):
            acc_v[pl.ds(k * 16, 16)] = acc_v[pl.ds(k * 16, 16)] * scale
        pltpu.sync_copy(acc_v, out_hbm)


@functools.cache
def _mean_kernel():
    return pl.kernel(
        _mean_body,
        out_type=jax.ShapeDtypeStruct((_D,), jnp.float32),
        mesh=plsc.VectorSubcoreMesh(core_axis_name="c", subcore_axis_name="s"),
        scratch_types=[
            pltpu.VMEM((2, _L // 2), jnp.int32),
            pltpu.VMEM((_L, _D), jnp.float32),
            pltpu.VMEM((_D,), jnp.float32),
            pltpu.SemaphoreType.DMA,
        ],
    )


def _logits_body(mean_ref, w_ref, b_ref, logits_ref, lse_ref, m_ref, s_ref):
    j = pl.program_id(0)

    @pl.when(j == 0)
    def _():
        m_ref[0] = -jnp.inf
        s_ref[0] = 0.0

    x = lax.dot_general(mean_ref[...], w_ref[...],
                        (((1,), (1,)), ((), ())),
                        preferred_element_type=jnp.float32)
    x = x + b_ref[0]
    logits_ref[0] = x

    m_old = m_ref[0]
    m_new = jnp.maximum(m_old, jnp.max(x))
    s_new = s_ref[0] * jnp.exp(m_old - m_new) + jnp.sum(jnp.exp(x - m_new))
    m_ref[0] = m_new
    s_ref[0] = s_new
    lse_ref[...] = jnp.reshape(m_new + jnp.log(s_new), (1, 1))


def _logits_call(mean2, W, b2):
    return pl.pallas_call(
        _logits_body,
        grid=(_NB,),
        in_specs=[
            pl.BlockSpec((1, _D), lambda j: (0, 0)),
            pl.BlockSpec((_BLK, _D), lambda j: (j, 0)),
            pl.BlockSpec((1, 1, _BLK), lambda j: (j, 0, 0)),
        ],
        out_specs=[
            pl.BlockSpec((1, 1, _BLK), lambda j: (j, 0, 0)),
            pl.BlockSpec((1, 1), lambda j: (0, 0)),
        ],
        out_shape=[
            jax.ShapeDtypeStruct((_NB, 1, _BLK), jnp.float32),
            jax.ShapeDtypeStruct((1, 1), jnp.float32),
        ],
        scratch_shapes=[
            pltpu.SMEM((1,), jnp.float32),
            pltpu.SMEM((1,), jnp.float32),
        ],
    )(mean2, W, b2)


def _sub_body(logits_ref, lse_ref, out_ref):
    out_ref[...] = logits_ref[...] - lse_ref[0, 0]


def _sub_call(logits, lse):
    return pl.pallas_call(
        _sub_body,
        in_specs=[
            pl.BlockSpec((_NB, 1, _BLK), lambda: (0, 0, 0)),
            pl.BlockSpec(memory_space=pltpu.SMEM),
        ],
        out_specs=pl.BlockSpec((_NB, 1, _BLK), lambda: (0, 0, 0)),
        out_shape=jax.ShapeDtypeStruct((_NB, 1, _BLK), jnp.float32),
    )(logits, lse)


def kernel(input, emb, W, b):
    idx = input.astype(jnp.int32).reshape(2, _L // 2)
    mean = _mean_kernel()(idx, emb)
    logits, lse = _logits_call(mean.reshape(1, _D), W, b.reshape(_NB, 1, _BLK))
    out = _sub_call(logits, lse)
    return out.reshape(1, _V)
